# baseline (speedup 1.0000x reference)
import jax
import jax.numpy as jnp
from jax.experimental import pallas as pl
from jax.experimental.pallas import tpu as pltpu

TILE_M = 512


def _fused_kernel(a_ref, x_ref, out_ref, a8_s, y1_s, recip_s, y2p_s):
    n = a8_s.shape[0]
    nt = n // TILE_M
    h = n // 2
    d = x_ref.shape[1]
    i = pl.program_id(0)

    @pl.when(i < nt)
    def _():
        a = a_ref[...]
        a8 = a.astype(jnp.int8)
        a8_s[pl.ds(i * TILE_M, TILE_M), :] = a8
        deg = jnp.maximum(jnp.sum(a, axis=1, keepdims=True), 1)
        r = 1.0 / deg.astype(jnp.float32)
        recip_s[pl.ds(i * TILE_M, TILE_M), :] = r
        af = a.astype(jnp.float32)
        y1 = jnp.dot(af, x_ref[...], preferred_element_type=jnp.float32) * r
        y1_s[pl.ds(i * TILE_M, TILE_M), :] = y1

    @pl.when(jnp.logical_and(i >= nt // 2, i < nt))
    def _():
        j = i - nt // 2
        afh = a8_s[pl.ds(j * TILE_M, TILE_M), 0:h].astype(jnp.float32)
        y2p_s[pl.ds(j * TILE_M, TILE_M), :] = jnp.dot(
            afh, y1_s[0:h, :], preferred_element_type=jnp.float32
        )

    @pl.when(i >= nt)
    def _():
        k = i - nt
        r = recip_s[pl.ds(k * TILE_M, TILE_M), :]

        @pl.when(k < nt // 2)
        def _():
            afh = a8_s[pl.ds(k * TILE_M, TILE_M), h:n].astype(jnp.float32)
            acc = jnp.dot(afh, y1_s[h:n, :], preferred_element_type=jnp.float32)
            out_ref[:, 3 * d:4 * d] = (
                y2p_s[pl.ds(k * TILE_M, TILE_M), :] + acc
            ) * r

        @pl.when(k >= nt // 2)
        def _():
            af = a8_s[pl.ds(k * TILE_M, TILE_M), :].astype(jnp.float32)
            out_ref[:, 3 * d:4 * d] = (
                jnp.dot(af, y1_s[...], preferred_element_type=jnp.float32) * r
            )

        y1f = y1_s[pl.ds(k * TILE_M, TILE_M), :]
        out_ref[:, 0:d] = x_ref[pl.ds(k * TILE_M, TILE_M), :]
        out_ref[:, d:2 * d] = y1f
        out_ref[:, 2 * d:3 * d] = y1f


def kernel(adjacency_matrix, node_features):
    n, d = node_features.shape
    nt = n // TILE_M

    out = pl.pallas_call(
        _fused_kernel,
        grid=(2 * nt,),
        in_specs=[
            pl.BlockSpec((TILE_M, n), lambda i: (jnp.minimum(i, nt - 1), 0)),
            pl.BlockSpec((n, d), lambda i: (0, 0)),
        ],
        out_specs=pl.BlockSpec(
            (TILE_M, 4 * d), lambda i: (jnp.maximum(i - nt, 0), 0)
        ),
        out_shape=jax.ShapeDtypeStruct((n, 4 * d), jnp.float32),
        scratch_shapes=[
            pltpu.VMEM((n, n), jnp.int8),
            pltpu.VMEM((n, d), jnp.float32),
            pltpu.VMEM((n, 1), jnp.float32),
            pltpu.VMEM((n // 2, d), jnp.float32),
        ],
        compiler_params=pltpu.CompilerParams(
            dimension_semantics=("arbitrary",),
            vmem_limit_bytes=64 * 1024 * 1024,
        ),
    )(adjacency_matrix, node_features)

    return out


# speedup vs baseline: 1.0164x; 1.0164x over previous
import jax
import jax.numpy as jnp
from jax.experimental import pallas as pl
from jax.experimental.pallas import tpu as pltpu

TILE_M = 512


def _fused_kernel(a_ref, x_ref, out_ref, a8_s, y1_s, recip_s):
    n = a8_s.shape[0]
    nt = n // TILE_M
    d = x_ref.shape[1]
    i = pl.program_id(0)

    @pl.when(i < nt)
    def _():
        a = a_ref[...]
        a8 = a.astype(jnp.int8)
        a8_s[pl.ds(i * TILE_M, TILE_M), :] = a8
        deg = jnp.maximum(jnp.sum(a, axis=1, keepdims=True), 1)
        r = 1.0 / deg.astype(jnp.float32)
        recip_s[pl.ds(i * TILE_M, TILE_M), :] = r
        af = a.astype(jnp.float32)
        y1 = jnp.dot(af, x_ref[...], preferred_element_type=jnp.float32) * r
        y1_s[pl.ds(i * TILE_M, TILE_M), :] = y1

    @pl.when(i >= nt)
    def _():
        k = i - nt
        af = a8_s[pl.ds(k * TILE_M, TILE_M), :].astype(jnp.float32)
        r = recip_s[pl.ds(k * TILE_M, TILE_M), :]
        y2 = jnp.dot(af, y1_s[...], preferred_element_type=jnp.float32) * r
        y1f = y1_s[pl.ds(k * TILE_M, TILE_M), :]
        out_ref[:, 0:d] = x_ref[pl.ds(k * TILE_M, TILE_M), :]
        out_ref[:, d:2 * d] = y1f
        out_ref[:, 2 * d:3 * d] = y1f
        out_ref[:, 3 * d:4 * d] = y2


def kernel(adjacency_matrix, node_features):
    n, d = node_features.shape
    nt = n // TILE_M

    out = pl.pallas_call(
        _fused_kernel,
        grid=(2 * nt,),
        in_specs=[
            pl.BlockSpec((TILE_M, n), lambda i: (jnp.minimum(i, nt - 1), 0)),
            pl.BlockSpec((n, d), lambda i: (0, 0)),
        ],
        out_specs=pl.BlockSpec(
            (TILE_M, 4 * d), lambda i: (jnp.maximum(i - nt, 0), 0)
        ),
        out_shape=jax.ShapeDtypeStruct((n, 4 * d), jnp.float32),
        scratch_shapes=[
            pltpu.VMEM((n, n), jnp.int8),
            pltpu.VMEM((n, d), jnp.float32),
            pltpu.VMEM((n, 1), jnp.float32),
        ],
        compiler_params=pltpu.CompilerParams(
            dimension_semantics=("arbitrary",),
        ),
    )(adjacency_matrix, node_features)

    return out
